# Initial kernel scaffold; baseline (speedup 1.0000x reference)
import jax
import jax.numpy as jnp
from jax.experimental import pallas as pl


def kernel(hidden, feature_bank, expert_bank_idx, ln_gamma, ln_beta, rW1, rb1, rW2, rb2, We1, be1, We2, be2, We3, be3, alpha):
    raise NotImplementedError("write your pallas kernel here")


# speedup vs baseline: 1.2415x; 1.2415x over previous
import jax
import jax.numpy as jnp
from jax.experimental import pallas as pl
from jax.experimental.pallas import tpu as pltpu

B = 2048
D = 2048
E = 8
NC = 16
FB = 16
FPE = 2
H = 1024
RH = 1024
K = 2

T = 256
MAX_TILES = (B * K) // T + (E - 1)
P = MAX_TILES * T


_SQRT_HALF = 0.7071067811865476


def _gelu(x):
    return 0.5 * x * (1.0 + jax.lax.erf(x * _SQRT_HALF))


def _router_body(hid_ref, sf_ref, g_ref, b_ref, w1h_ref, w1f_ref, b1_ref,
                 w2_ref, b2_ref, h_out_ref, r4_ref):
    x = hid_ref[...]
    mu = jnp.mean(x, axis=1, keepdims=True)
    xc = x - mu
    var = jnp.mean(xc * xc, axis=1, keepdims=True)
    h = xc * jax.lax.rsqrt(var + 1e-5) * g_ref[...] + b_ref[...]
    h_out_ref[...] = h
    t1 = jnp.dot(h, w1h_ref[...], preferred_element_type=jnp.float32)
    t1 = t1 + jnp.dot(sf_ref[...], w1f_ref[...],
                      preferred_element_type=jnp.float32)
    t1 = _gelu(t1 + b1_ref[...])
    logits = jnp.dot(t1, w2_ref[...],
                     preferred_element_type=jnp.float32) + b2_ref[...]
    i1 = jnp.argmax(logits, axis=1)
    v1 = jnp.max(logits, axis=1, keepdims=True)
    masked = jnp.where(jnp.arange(E)[None, :] == i1[:, None],
                       -jnp.inf, logits)
    i2 = jnp.argmax(masked, axis=1)
    v2 = jnp.max(masked, axis=1, keepdims=True)
    e2 = jnp.exp(v2 - v1)
    w1 = 1.0 / (1.0 + e2)
    w2 = e2 * w1
    r4_ref[...] = jnp.concatenate(
        [i1[:, None].astype(jnp.float32), i2[:, None].astype(jnp.float32),
         w1, w2], axis=1)


def _run_router(hidden, stage_feats, ln_gamma, ln_beta, rW1, rb1, rW2, rb2):
    TB = 256
    grid = (B // TB,)
    w1h = rW1[:D]
    w1f = rW1[D:]
    h_ln, r4 = pl.pallas_call(
        _router_body,
        grid=grid,
        in_specs=[
            pl.BlockSpec((TB, D), lambda i: (i, 0)),
            pl.BlockSpec((TB, NC * FB), lambda i: (i, 0)),
            pl.BlockSpec((1, D), lambda i: (0, 0)),
            pl.BlockSpec((1, D), lambda i: (0, 0)),
            pl.BlockSpec((D, RH), lambda i: (0, 0)),
            pl.BlockSpec((NC * FB, RH), lambda i: (0, 0)),
            pl.BlockSpec((1, RH), lambda i: (0, 0)),
            pl.BlockSpec((RH, E), lambda i: (0, 0)),
            pl.BlockSpec((1, E), lambda i: (0, 0)),
        ],
        out_specs=[
            pl.BlockSpec((TB, D), lambda i: (i, 0)),
            pl.BlockSpec((TB, 4), lambda i: (i, 0)),
        ],
        out_shape=[
            jax.ShapeDtypeStruct((B, D), jnp.float32),
            jax.ShapeDtypeStruct((B, 4), jnp.float32),
        ],
    )(hidden, stage_feats, ln_gamma.reshape(1, D), ln_beta.reshape(1, D),
      w1h, w1f, rb1.reshape(1, RH), rW2, rb2.reshape(1, E))
    return h_ln, r4


def _gemm_body(te_ref, xh_ref, xf_ref, w_ref, w1h_ref, w1f_ref, b1_ref,
               w2_ref, b2_ref, w3_ref, b3_ref, out_ref):
    a = jnp.dot(xh_ref[...], w1h_ref[0], preferred_element_type=jnp.float32)
    a = a + jnp.dot(xf_ref[...], w1f_ref[0],
                    preferred_element_type=jnp.float32)
    a = _gelu(a + b1_ref[0])
    h2 = _gelu(jnp.dot(a, w2_ref[0],
                       preferred_element_type=jnp.float32) + b2_ref[0])
    o = jnp.dot(h2, w3_ref[0], preferred_element_type=jnp.float32)
    out_ref[...] = (o + b3_ref[0]) * w_ref[...]


def _run_gemm(tile_e, xh, xf, w_pad, We1, be1, We2, be2, We3, be3):
    F2 = FPE * FB
    w1h = We1[:, :D, :]
    w1f = We1[:, D:, :]
    grid_spec = pltpu.PrefetchScalarGridSpec(
        num_scalar_prefetch=1,
        grid=(MAX_TILES,),
        in_specs=[
            pl.BlockSpec((T, D), lambda i, s: (i, 0)),
            pl.BlockSpec((T, F2), lambda i, s: (i, 0)),
            pl.BlockSpec((T, 1), lambda i, s: (i, 0)),
            pl.BlockSpec((1, D, H), lambda i, s: (s[i], 0, 0)),
            pl.BlockSpec((1, F2, H), lambda i, s: (s[i], 0, 0)),
            pl.BlockSpec((1, 1, H), lambda i, s: (s[i], 0, 0)),
            pl.BlockSpec((1, H, H), lambda i, s: (s[i], 0, 0)),
            pl.BlockSpec((1, 1, H), lambda i, s: (s[i], 0, 0)),
            pl.BlockSpec((1, H, D), lambda i, s: (s[i], 0, 0)),
            pl.BlockSpec((1, 1, D), lambda i, s: (s[i], 0, 0)),
        ],
        out_specs=pl.BlockSpec((T, D), lambda i, s: (i, 0)),
    )
    out_pad = pl.pallas_call(
        _gemm_body,
        grid_spec=grid_spec,
        out_shape=jax.ShapeDtypeStruct((P, D), jnp.float32),
    )(tile_e, xh, xf, w_pad, w1h, w1f, be1.reshape(E, 1, H), We2,
      be2.reshape(E, 1, H), We3, be3.reshape(E, 1, D))
    return out_pad


def kernel(hidden, feature_bank, expert_bank_idx, ln_gamma, ln_beta,
           rW1, rb1, rW2, rb2, We1, be1, We2, be2, We3, be3, alpha):
    stage_feats = feature_bank.reshape(B, NC * FB)
    h_ln, r4 = _run_router(hidden, stage_feats, ln_gamma, ln_beta,
                           rW1, rb1, rW2, rb2)

    i1 = r4[:, 0].astype(jnp.int32)
    i2 = r4[:, 1].astype(jnp.int32)
    e_pair = jnp.stack([i1, i2], axis=1).reshape(-1)
    w_pair = (r4[:, 2:4] * alpha).reshape(-1)
    oh = (e_pair[:, None] == jnp.arange(E)[None, :]).astype(jnp.int32)
    ranks = jnp.cumsum(oh, axis=0)
    rank_in = jnp.take_along_axis(ranks, e_pair[:, None], axis=1)[:, 0] - 1
    counts = ranks[-1]
    tiles_pe = (counts + T - 1) // T
    tile_end = jnp.cumsum(tiles_pe)
    pad_start = (tile_end - tiles_pe) * T
    pos = pad_start[e_pair] + rank_in
    tok_pair = jnp.repeat(jnp.arange(B, dtype=jnp.int32), K)
    stok = jnp.zeros((P,), jnp.int32).at[pos].set(tok_pair)
    w_pad = jnp.zeros((P,), jnp.float32).at[pos].set(w_pair)
    tile_e = jnp.minimum(
        jnp.sum(jnp.arange(MAX_TILES)[:, None] >= tile_end[None, :], axis=1),
        E - 1).astype(jnp.int32)
    row_e = jnp.repeat(tile_e, T)
    p1 = pos[0::2]
    p2 = pos[1::2]

    fb16 = feature_bank.reshape(B * NC, FB)
    c0 = expert_bank_idx[row_e, 0].astype(jnp.int32)
    c1 = expert_bank_idx[row_e, 1].astype(jnp.int32)
    xh = h_ln[stok]
    xf = jnp.concatenate([fb16[stok * NC + c0], fb16[stok * NC + c1]],
                         axis=1)

    out_pad = _run_gemm(tile_e, xh, xf, w_pad.reshape(P, 1),
                        We1, be1, We2, be2, We3, be3)

    y = hidden + out_pad[p1] + out_pad[p2]
    return y
